# baseline (speedup 1.0000x reference)
import jax
import jax.numpy as jnp
from jax import lax
from jax.experimental import pallas as pl
from jax.experimental.pallas import tpu as pltpu
from jax.experimental.pallas import tpu_sc as plsc

N_CELL = 100000
N_TISSUE = 2000
C = 50
D = 128

NW = 32
SPW = 3136
NPAD = NW * SPW
CH = 112
NCH = SPW // CH
TAB = 2056
ZROW = 2048
TCHUNK = 200


def _sc1_body(ca3d, t3d, cell_hbm, ca64f, neg1, ztab,
              agg_hbm, tmap_hbm,
              idx_a, tix_a, rows_a, flat_v, buf_v, shared, gsem):
    c = lax.axis_index("c")
    s = lax.axis_index("s")
    w = c * 16 + s

    soff = pl.multiple_of(s * 128, 8)
    pltpu.sync_copy(ztab.at[pl.ds(soff, 128)], shared.at[pl.ds(soff, 128)])

    @pl.when(s == 15)
    def _():
        pltpu.sync_copy(ztab.at[pl.ds(2048, 8)], shared.at[pl.ds(2048, 8)])

    plsc.subcore_barrier()

    pltpu.sync_copy(ca3d.at[w], idx_a)
    pltpu.sync_copy(t3d.at[w], tix_a)

    cp0 = pltpu.async_copy(cell_hbm.at[idx_a.at[0]], rows_a.at[0], gsem.at[0])

    def chunk_a(k, _):
        sl = lax.rem(k, 2)
        nsl = lax.rem(k + 1, 2)

        @pl.when(k + 1 < NCH)
        def _():
            pltpu.async_copy(cell_hbm.at[idx_a.at[k + 1]], rows_a.at[nsl],
                             gsem.at[nsl])

        pltpu.make_async_copy(cell_hbm.at[idx_a.at[k]], rows_a.at[sl],
                              gsem.at[sl]).wait()
        pltpu.sync_copy(rows_a.at[sl], shared.at[tix_a.at[k]], add=True)
        return 0

    lax.fori_loop(0, NCH, chunk_a, 0)
    del cp0
    plsc.subcore_barrier()

    @pl.when(s < 15)
    def _():
        pltpu.sync_copy(shared.at[pl.ds(soff, 128)],
                        agg_hbm.at[c, pl.ds(soff, 128)])

    @pl.when(s == 15)
    def _():
        pltpu.sync_copy(shared.at[pl.ds(1920, 80)],
                        agg_hbm.at[c, pl.ds(1920, 80)])

    pltpu.sync_copy(neg1, buf_v)
    cbase = w * SPW

    lanes = lax.iota(jnp.int32, 16)

    def tblock(rc, tv0):
        roff = pl.multiple_of(rc * TCHUNK * 64, 16)
        pltpu.sync_copy(ca64f.at[pl.ds(roff, TCHUNK * 64)], flat_v)

        def trow(j, carry):
            tv, ivs = carry
            for k in range(4):
                v = plsc.load_gather(flat_v, [ivs[k]])
                d = v - cbase
                m = (d >= 0) & (d < SPW)
                plsc.store_scatter(buf_v, [d], tv, mask=m)
            return tv + 1, tuple(iv + 64 for iv in ivs)

        ivs0 = tuple(lanes + k * 16 for k in range(4))
        tv1, _ = lax.fori_loop(0, TCHUNK, trow, (tv0, ivs0))
        return tv1

    tv0 = jnp.zeros((16,), jnp.int32)
    lax.fori_loop(0, N_TISSUE // TCHUNK, tblock, tv0)
    pltpu.sync_copy(buf_v, tmap_hbm.at[pl.ds(pl.multiple_of(w * SPW, 16), SPW)])


def _sc1(ca3d, t3d, cell_graph, ca64f, neg1, ztab):
    mesh = plsc.VectorSubcoreMesh(core_axis_name="c", subcore_axis_name="s")
    return pl.kernel(
        _sc1_body,
        out_type=[
            jax.ShapeDtypeStruct((2, N_TISSUE, D), jnp.float32),
            jax.ShapeDtypeStruct((NPAD,), jnp.int32),
        ],
        mesh=mesh,
        scratch_types=[
            pltpu.VMEM((NCH, CH), jnp.int32),
            pltpu.VMEM((NCH, CH), jnp.int32),
            pltpu.VMEM((2, CH, D), jnp.float32),
            pltpu.VMEM((TCHUNK * 64,), jnp.int32),
            pltpu.VMEM((SPW,), jnp.int32),
            pltpu.VMEM_SHARED((TAB, D), jnp.float32),
            pltpu.SemaphoreType.DMA((2,)),
        ],
        compiler_params=pltpu.CompilerParams(needs_layout_passes=False, use_tc_tiling_on_sc=False),
    )(ca3d, t3d, cell_graph, ca64f, neg1, ztab)


def _sc2_body(tmap_hbm, table_hbm, tfeat_hbm, idx_v, rows_b, sems):
    c = lax.axis_index("c")
    s = lax.axis_index("s")
    w = c * 16 + s

    pltpu.sync_copy(tmap_hbm.at[pl.ds(pl.multiple_of(w * SPW, 16), SPW)], idx_v)

    tb = w * TAB

    def clampb(h, iv):
        v = plsc.load_gather(idx_v, [iv])
        plsc.store_scatter(idx_v, [iv], jnp.where(v < 0, ZROW, v) + tb)
        return iv + 16

    lax.fori_loop(0, SPW // 16, clampb, lax.iota(jnp.int32, 16))

    cp0 = pltpu.async_copy(table_hbm.at[idx_v.at[pl.ds(0, CH)]], rows_b.at[0],
                           sems.at[0])

    def chunk(k, _):
        sl = lax.rem(k, 2)

        @pl.when(k + 1 < NCH)
        def _():
            nsl = lax.rem(k + 1, 2)
            io = pl.multiple_of((k + 1) * CH, 16)
            pltpu.async_copy(table_hbm.at[idx_v.at[pl.ds(io, CH)]],
                             rows_b.at[nsl], sems.at[nsl])

        io2 = pl.multiple_of(k * CH, 16)
        pltpu.make_async_copy(table_hbm.at[idx_v.at[pl.ds(io2, CH)]],
                              rows_b.at[sl], sems.at[sl]).wait()
        base = pl.multiple_of(w * SPW + k * CH, 16)
        pltpu.sync_copy(rows_b.at[sl], tfeat_hbm.at[pl.ds(base, CH)])
        return 0

    lax.fori_loop(0, NCH, chunk, 0)
    del cp0


def _sc2(tmap, table32):
    mesh = plsc.VectorSubcoreMesh(core_axis_name="c", subcore_axis_name="s")
    return pl.kernel(
        _sc2_body,
        out_type=jax.ShapeDtypeStruct((NPAD, D), jnp.bfloat16),
        mesh=mesh,
        scratch_types=[
            pltpu.VMEM((SPW,), jnp.int32),
            pltpu.VMEM((2, CH, D), jnp.bfloat16),
            pltpu.SemaphoreType.DMA((2,)),
        ],
        compiler_params=pltpu.CompilerParams(needs_layout_passes=False, use_tc_tiling_on_sc=False),
    )(tmap, table32)


def _tc1_body(tg_ref, agg_ref, out_ref):
    tu = tg_ref[...] + agg_ref[0] + agg_ref[1]
    out_ref[pl.ds(0, N_TISSUE), :] = tu
    out_ref[pl.ds(N_TISSUE, TAB - N_TISSUE), :] = jnp.zeros(
        (TAB - N_TISSUE, D), jnp.float32)


def _rep_body(tab_ref, out_ref):
    out_ref[...] = tab_ref[...]


def _tc2_body(cell_ref, tf_ref, wcq_ref, wtp_ref, b_ref, atd_ref, out_ref):
    cell = cell_ref[...]
    tf16 = tf_ref[...]
    cq = jnp.dot(cell.astype(jnp.bfloat16), wcq_ref[...],
                 preferred_element_type=jnp.float32)
    tk = jnp.dot(tf16, wtp_ref[...], preferred_element_type=jnp.float32)
    h = jnp.tanh(cq + tk + b_ref[...])
    score = jnp.sum(h * atd_ref[...], axis=1, keepdims=True)
    wgt = jax.nn.sigmoid(score)
    out_ref[...] = cell + tf16.astype(jnp.float32) * wgt


def kernel(cell_graph, tissue_graph, cluster_assignments,
           W_cp, b_cp, W_tq, b_tq, a_bu,
           W_tp, b_tp, W_cq, b_cq, a_td):
    ca_flat = cluster_assignments.reshape(-1)
    ca_flat_pad = jnp.concatenate(
        [ca_flat, jnp.zeros((NPAD - N_CELL,), jnp.int32)])
    ca3d = ca_flat_pad.reshape(NW, NCH, CH)
    t_flat = jnp.repeat(jnp.arange(N_TISSUE, dtype=jnp.int32), C)
    t3d = jnp.concatenate(
        [t_flat, jnp.full((NPAD - N_CELL,), ZROW, jnp.int32)]).reshape(
            NW, NCH, CH)
    ca64f = jnp.pad(cluster_assignments, ((0, 0), (0, 64 - C)),
                    constant_values=-1).reshape(-1)
    neg1 = jnp.full((SPW,), -1, jnp.int32)
    ztab = jnp.zeros((TAB, D), jnp.float32)

    agg, tmap = _sc1(ca3d, t3d, cell_graph, ca64f, neg1, ztab)

    table = pl.pallas_call(
        _tc1_body,
        out_shape=jax.ShapeDtypeStruct((TAB, D), jnp.float32),
    )(tissue_graph, agg)

    table32 = pl.pallas_call(
        _rep_body,
        grid=(NW,),
        in_specs=[pl.BlockSpec((TAB, D), lambda i: (0, 0))],
        out_specs=pl.BlockSpec((TAB, D), lambda i: (i, 0)),
        out_shape=jax.ShapeDtypeStruct((NW * TAB, D), jnp.bfloat16),
    )(table.astype(jnp.bfloat16))

    tfeat = _sc2(tmap, table32)

    R = 1000
    bsum = (b_cq + b_tp).reshape(1, D)
    cell_updated = pl.pallas_call(
        _tc2_body,
        grid=(N_CELL // R,),
        in_specs=[
            pl.BlockSpec((R, D), lambda i: (i, 0)),
            pl.BlockSpec((R, D), lambda i: (i, 0)),
            pl.BlockSpec((D, D), lambda i: (0, 0)),
            pl.BlockSpec((D, D), lambda i: (0, 0)),
            pl.BlockSpec((1, D), lambda i: (0, 0)),
            pl.BlockSpec((1, D), lambda i: (0, 0)),
        ],
        out_specs=pl.BlockSpec((R, D), lambda i: (i, 0)),
        out_shape=jax.ShapeDtypeStruct((N_CELL, D), jnp.float32),
    )(cell_graph, tfeat, W_cq.T.astype(jnp.bfloat16),
      W_tp.T.astype(jnp.bfloat16), bsum, a_td)

    tissue_updated = table[:N_TISSUE]
    return (cell_updated, tissue_updated)


# speedup vs baseline: 1.2087x; 1.2087x over previous
import jax
import jax.numpy as jnp
from jax import lax
from jax.experimental import pallas as pl
from jax.experimental.pallas import tpu as pltpu
from jax.experimental.pallas import tpu_sc as plsc

N_CELL = 100000
N_TISSUE = 2000
C = 50
D = 128

NW = 32
SPW = 3136
NPAD = NW * SPW
CH = 112
NCH = SPW // CH
TAB = 2056
ZROW = 2048
TCHUNK = 200


def _sc1_body(ca3d, t3d, cell_hbm, ca64f, neg1, ztab,
              agg_hbm, tmap_hbm,
              idx_a, tix_a, rows_a, flat_v, buf_v, shared, gsem):
    c = lax.axis_index("c")
    s = lax.axis_index("s")
    w = c * 16 + s

    soff = pl.multiple_of(s * 128, 8)
    pltpu.sync_copy(ztab.at[pl.ds(soff, 128)], shared.at[pl.ds(soff, 128)])

    @pl.when(s == 15)
    def _():
        pltpu.sync_copy(ztab.at[pl.ds(2048, 8)], shared.at[pl.ds(2048, 8)])

    plsc.subcore_barrier()

    pltpu.sync_copy(ca3d.at[w], idx_a)
    pltpu.sync_copy(t3d.at[w], tix_a)

    cp0 = pltpu.async_copy(cell_hbm.at[idx_a.at[0]], rows_a.at[0], gsem.at[0])

    def chunk_a(k, _):
        sl = lax.rem(k, 2)
        nsl = lax.rem(k + 1, 2)

        @pl.when(k + 1 < NCH)
        def _():
            pltpu.async_copy(cell_hbm.at[idx_a.at[k + 1]], rows_a.at[nsl],
                             gsem.at[nsl])

        pltpu.make_async_copy(cell_hbm.at[idx_a.at[k]], rows_a.at[sl],
                              gsem.at[sl]).wait()
        pltpu.sync_copy(rows_a.at[sl], shared.at[tix_a.at[k]], add=True)
        return 0

    lax.fori_loop(0, NCH, chunk_a, 0)
    del cp0
    plsc.subcore_barrier()

    @pl.when(s < 15)
    def _():
        pltpu.sync_copy(shared.at[pl.ds(soff, 128)],
                        agg_hbm.at[c, pl.ds(soff, 128)])

    @pl.when(s == 15)
    def _():
        pltpu.sync_copy(shared.at[pl.ds(1920, 80)],
                        agg_hbm.at[c, pl.ds(1920, 80)])

    pltpu.sync_copy(neg1, buf_v)
    cbase = w * SPW

    lanes = lax.iota(jnp.int32, 16)

    def tblock(rc, tv0):
        roff = pl.multiple_of(rc * TCHUNK * 64, 16)
        pltpu.sync_copy(ca64f.at[pl.ds(roff, TCHUNK * 64)], flat_v)

        def trow(j, carry):
            tv, ivs = carry
            for k in range(4):
                v = plsc.load_gather(flat_v, [ivs[k]])
                d = v - cbase
                m = (d >= 0) & (d < SPW)
                plsc.store_scatter(buf_v, [d], tv, mask=m)
            return tv + 1, tuple(iv + 64 for iv in ivs)

        ivs0 = tuple(lanes + k * 16 for k in range(4))
        tv1, _ = lax.fori_loop(0, TCHUNK, trow, (tv0, ivs0))
        return tv1

    tv0 = jnp.zeros((16,), jnp.int32)
    lax.fori_loop(0, N_TISSUE // TCHUNK, tblock, tv0)
    pltpu.sync_copy(buf_v, tmap_hbm.at[pl.ds(pl.multiple_of(w * SPW, 16), SPW)])


def _sc1(ca3d, t3d, cell_graph, ca64f, neg1, ztab):
    mesh = plsc.VectorSubcoreMesh(core_axis_name="c", subcore_axis_name="s")
    return pl.kernel(
        _sc1_body,
        out_type=[
            jax.ShapeDtypeStruct((2, N_TISSUE, D), jnp.float32),
            jax.ShapeDtypeStruct((NPAD,), jnp.int32),
        ],
        mesh=mesh,
        scratch_types=[
            pltpu.VMEM((NCH, CH), jnp.int32),
            pltpu.VMEM((NCH, CH), jnp.int32),
            pltpu.VMEM((2, CH, D), jnp.float32),
            pltpu.VMEM((TCHUNK * 64,), jnp.int32),
            pltpu.VMEM((SPW,), jnp.int32),
            pltpu.VMEM_SHARED((TAB, D), jnp.float32),
            pltpu.SemaphoreType.DMA((2,)),
        ],
        compiler_params=pltpu.CompilerParams(needs_layout_passes=False, use_tc_tiling_on_sc=False),
    )(ca3d, t3d, cell_graph, ca64f, neg1, ztab)


def _sc2_body(tmap_hbm, table_hbm, tfeat_hbm, idx_v, rows_b, sems):
    c = lax.axis_index("c")
    s = lax.axis_index("s")
    w = c * 16 + s

    pltpu.sync_copy(tmap_hbm.at[pl.ds(pl.multiple_of(w * SPW, 16), SPW)], idx_v)

    tb = w * TAB

    def clampb(h, iv):
        v = plsc.load_gather(idx_v, [iv])
        plsc.store_scatter(idx_v, [iv], jnp.where(v < 0, ZROW, v) + tb)
        return iv + 16

    lax.fori_loop(0, SPW // 16, clampb, lax.iota(jnp.int32, 16))

    cp0 = pltpu.async_copy(table_hbm.at[idx_v.at[pl.ds(0, CH)]], rows_b.at[0],
                           sems.at[0])

    def chunk(k, _):
        sl = lax.rem(k, 2)

        @pl.when(k + 1 < NCH)
        def _():
            nsl = lax.rem(k + 1, 2)
            io = pl.multiple_of((k + 1) * CH, 16)
            pltpu.async_copy(table_hbm.at[idx_v.at[pl.ds(io, CH)]],
                             rows_b.at[nsl], sems.at[nsl])

        io2 = pl.multiple_of(k * CH, 16)
        pltpu.make_async_copy(table_hbm.at[idx_v.at[pl.ds(io2, CH)]],
                              rows_b.at[sl], sems.at[sl]).wait()
        base = pl.multiple_of(w * SPW + k * CH, 16)
        pltpu.sync_copy(rows_b.at[sl], tfeat_hbm.at[pl.ds(base, CH)])
        return 0

    lax.fori_loop(0, NCH, chunk, 0)
    del cp0


def _sc2(tmap, table32):
    mesh = plsc.VectorSubcoreMesh(core_axis_name="c", subcore_axis_name="s")
    return pl.kernel(
        _sc2_body,
        out_type=jax.ShapeDtypeStruct((NPAD, D), jnp.float32),
        mesh=mesh,
        scratch_types=[
            pltpu.VMEM((SPW,), jnp.int32),
            pltpu.VMEM((2, CH, D), jnp.float32),
            pltpu.SemaphoreType.DMA((2,)),
        ],
        compiler_params=pltpu.CompilerParams(needs_layout_passes=False, use_tc_tiling_on_sc=False),
    )(tmap, table32)


def _tc1_body(tg_ref, agg_ref, out_ref):
    tu = tg_ref[...] + agg_ref[0] + agg_ref[1]
    out_ref[pl.ds(0, N_TISSUE), :] = tu
    out_ref[pl.ds(N_TISSUE, TAB - N_TISSUE), :] = jnp.zeros(
        (TAB - N_TISSUE, D), jnp.float32)


def _rep_body(tab_ref, out_ref):
    out_ref[...] = tab_ref[...]


def _tc2_body(cell_ref, tf_ref, wcq_ref, wtp_ref, b_ref, atd_ref, out_ref):
    cell = cell_ref[...]
    tf = tf_ref[...]
    cq = jnp.dot(cell.astype(jnp.bfloat16), wcq_ref[...],
                 preferred_element_type=jnp.float32)
    tk = jnp.dot(tf.astype(jnp.bfloat16), wtp_ref[...],
                 preferred_element_type=jnp.float32)
    h = jnp.tanh(cq + tk + b_ref[...])
    score = jnp.sum(h * atd_ref[...], axis=1, keepdims=True)
    wgt = jax.nn.sigmoid(score)
    out_ref[...] = cell + tf * wgt


def kernel(cell_graph, tissue_graph, cluster_assignments,
           W_cp, b_cp, W_tq, b_tq, a_bu,
           W_tp, b_tp, W_cq, b_cq, a_td):
    ca_flat = cluster_assignments.reshape(-1)
    ca_flat_pad = jnp.concatenate(
        [ca_flat, jnp.zeros((NPAD - N_CELL,), jnp.int32)])
    ca3d = ca_flat_pad.reshape(NW, NCH, CH)
    t_flat = jnp.repeat(jnp.arange(N_TISSUE, dtype=jnp.int32), C)
    t3d = jnp.concatenate(
        [t_flat, jnp.full((NPAD - N_CELL,), ZROW, jnp.int32)]).reshape(
            NW, NCH, CH)
    ca64f = jnp.pad(cluster_assignments, ((0, 0), (0, 64 - C)),
                    constant_values=-1).reshape(-1)
    neg1 = jnp.full((SPW,), -1, jnp.int32)
    ztab = jnp.zeros((TAB, D), jnp.float32)

    agg, tmap = _sc1(ca3d, t3d, cell_graph, ca64f, neg1, ztab)

    table = pl.pallas_call(
        _tc1_body,
        out_shape=jax.ShapeDtypeStruct((TAB, D), jnp.float32),
    )(tissue_graph, agg)

    table32 = pl.pallas_call(
        _rep_body,
        grid=(NW,),
        in_specs=[pl.BlockSpec((TAB, D), lambda i: (0, 0))],
        out_specs=pl.BlockSpec((TAB, D), lambda i: (i, 0)),
        out_shape=jax.ShapeDtypeStruct((NW * TAB, D), jnp.float32),
    )(table)

    tfeat = _sc2(tmap, table32)

    R = 1000
    bsum = (b_cq + b_tp).reshape(1, D)
    cell_updated = pl.pallas_call(
        _tc2_body,
        grid=(N_CELL // R,),
        in_specs=[
            pl.BlockSpec((R, D), lambda i: (i, 0)),
            pl.BlockSpec((R, D), lambda i: (i, 0)),
            pl.BlockSpec((D, D), lambda i: (0, 0)),
            pl.BlockSpec((D, D), lambda i: (0, 0)),
            pl.BlockSpec((1, D), lambda i: (0, 0)),
            pl.BlockSpec((1, D), lambda i: (0, 0)),
        ],
        out_specs=pl.BlockSpec((R, D), lambda i: (i, 0)),
        out_shape=jax.ShapeDtypeStruct((N_CELL, D), jnp.float32),
    )(cell_graph, tfeat, W_cq.T.astype(jnp.bfloat16),
      W_tp.T.astype(jnp.bfloat16), bsum, a_td)

    tissue_updated = table[:N_TISSUE]
    return (cell_updated, tissue_updated)


# speedup vs baseline: 1.4098x; 1.1664x over previous
import jax
import jax.numpy as jnp
from jax import lax
from jax.experimental import pallas as pl
from jax.experimental.pallas import tpu as pltpu
from jax.experimental.pallas import tpu_sc as plsc

N_CELL = 100000
N_TISSUE = 2000
C = 50
D = 128

NW = 32
SPW = 3136
NPAD = NW * SPW
CH = 112
NCH = SPW // CH
TAB = 2056
ZROW = 2048
TCHUNK = 200


def _sc1_body(ca3d, t3d, cell_hbm, ca64f, neg1, ztab,
              agg_hbm, tmap_hbm,
              idx_a, tix_a, rows_a, flat_v, buf_v, shared, gsem):
    c = lax.axis_index("c")
    s = lax.axis_index("s")
    w = c * 16 + s

    soff = pl.multiple_of(s * 128, 8)
    pltpu.sync_copy(ztab.at[pl.ds(soff, 128)], shared.at[pl.ds(soff, 128)])

    @pl.when(s == 15)
    def _():
        pltpu.sync_copy(ztab.at[pl.ds(2048, 8)], shared.at[pl.ds(2048, 8)])

    plsc.subcore_barrier()

    pltpu.sync_copy(ca3d.at[w], idx_a)
    pltpu.sync_copy(t3d.at[w], tix_a)

    cp0 = pltpu.async_copy(cell_hbm.at[idx_a.at[0]], rows_a.at[0], gsem.at[0])
    cp1 = pltpu.async_copy(cell_hbm.at[idx_a.at[1]], rows_a.at[1], gsem.at[1])

    def chunk_a(k, _):
        sl = lax.rem(k, 3)

        @pl.when(k + 2 < NCH)
        def _():
            nsl = lax.rem(k + 2, 3)
            pltpu.async_copy(cell_hbm.at[idx_a.at[k + 2]], rows_a.at[nsl],
                             gsem.at[nsl])

        pltpu.make_async_copy(cell_hbm.at[idx_a.at[k]], rows_a.at[sl],
                              gsem.at[sl]).wait()
        pltpu.sync_copy(rows_a.at[sl], shared.at[tix_a.at[k]], add=True)
        return 0

    lax.fori_loop(0, NCH, chunk_a, 0)
    del cp0, cp1
    plsc.subcore_barrier()

    @pl.when(s < 15)
    def _():
        pltpu.sync_copy(shared.at[pl.ds(soff, 128)],
                        agg_hbm.at[c, pl.ds(soff, 128)])

    @pl.when(s == 15)
    def _():
        pltpu.sync_copy(shared.at[pl.ds(1920, 80)],
                        agg_hbm.at[c, pl.ds(1920, 80)])

    pltpu.sync_copy(neg1, buf_v)
    cbase = w * SPW

    lanes = lax.iota(jnp.int32, 16)

    def tblock(rc, tv0):
        roff = pl.multiple_of(rc * TCHUNK * 64, 16)
        pltpu.sync_copy(ca64f.at[pl.ds(roff, TCHUNK * 64)], flat_v)

        def trow(j, carry):
            tv, ivs = carry
            vs = [plsc.load_gather(flat_v, [ivs[k]]) for k in range(4)]
            ds_ = [v - cbase for v in vs]
            ms = [(d >= 0) & (d < SPW) for d in ds_]
            for k in range(4):
                plsc.store_scatter(buf_v, [ds_[k]], tv, mask=ms[k])
            return tv + 1, tuple(iv + 64 for iv in ivs)

        ivs0 = tuple(lanes + k * 16 for k in range(4))
        tv1, _ = lax.fori_loop(0, TCHUNK, trow, (tv0, ivs0))
        return tv1

    tv0 = jnp.zeros((16,), jnp.int32)
    lax.fori_loop(0, N_TISSUE // TCHUNK, tblock, tv0)
    pltpu.sync_copy(buf_v, tmap_hbm.at[pl.ds(pl.multiple_of(w * SPW, 16), SPW)])


def _sc1(ca3d, t3d, cell_graph, ca64f, neg1, ztab):
    mesh = plsc.VectorSubcoreMesh(core_axis_name="c", subcore_axis_name="s")
    return pl.kernel(
        _sc1_body,
        out_type=[
            jax.ShapeDtypeStruct((2, N_TISSUE, D), jnp.float32),
            jax.ShapeDtypeStruct((NPAD,), jnp.int32),
        ],
        mesh=mesh,
        scratch_types=[
            pltpu.VMEM((NCH, CH), jnp.int32),
            pltpu.VMEM((NCH, CH), jnp.int32),
            pltpu.VMEM((3, CH, D), jnp.float32),
            pltpu.VMEM((TCHUNK * 64,), jnp.int32),
            pltpu.VMEM((SPW,), jnp.int32),
            pltpu.VMEM_SHARED((TAB, D), jnp.float32),
            pltpu.SemaphoreType.DMA((3,)),
        ],
        compiler_params=pltpu.CompilerParams(needs_layout_passes=False, use_tc_tiling_on_sc=False),
    )(ca3d, t3d, cell_graph, ca64f, neg1, ztab)


def _sc2_body(tmap_hbm, table_hbm, tfeat_hbm, idx_v, rows_b, sems):
    c = lax.axis_index("c")
    s = lax.axis_index("s")
    w = c * 16 + s

    pltpu.sync_copy(tmap_hbm.at[pl.ds(pl.multiple_of(w * SPW, 16), SPW)], idx_v)

    tb = w * TAB

    def clampb(h, iv):
        v = plsc.load_gather(idx_v, [iv])
        plsc.store_scatter(idx_v, [iv], jnp.where(v < 0, ZROW, v) + tb)
        return iv + 16

    lax.fori_loop(0, SPW // 16, clampb, lax.iota(jnp.int32, 16))

    cp0 = pltpu.async_copy(table_hbm.at[idx_v.at[pl.ds(0, CH)]], rows_b.at[0],
                           sems.at[0])
    cp1 = pltpu.async_copy(table_hbm.at[idx_v.at[pl.ds(CH, CH)]], rows_b.at[1],
                           sems.at[1])

    def chunk(k, _):
        sl = lax.rem(k, 3)

        @pl.when(k + 2 < NCH)
        def _():
            nsl = lax.rem(k + 2, 3)
            io = pl.multiple_of((k + 2) * CH, 16)
            pltpu.async_copy(table_hbm.at[idx_v.at[pl.ds(io, CH)]],
                             rows_b.at[nsl], sems.at[nsl])

        io2 = pl.multiple_of(k * CH, 16)
        pltpu.make_async_copy(table_hbm.at[idx_v.at[pl.ds(io2, CH)]],
                              rows_b.at[sl], sems.at[sl]).wait()
        base = pl.multiple_of(w * SPW + k * CH, 16)
        pltpu.sync_copy(rows_b.at[sl], tfeat_hbm.at[pl.ds(base, CH)])
        return 0

    lax.fori_loop(0, NCH, chunk, 0)
    del cp0, cp1


def _sc2(tmap, table32):
    mesh = plsc.VectorSubcoreMesh(core_axis_name="c", subcore_axis_name="s")
    return pl.kernel(
        _sc2_body,
        out_type=jax.ShapeDtypeStruct((NPAD, D), jnp.float32),
        mesh=mesh,
        scratch_types=[
            pltpu.VMEM((SPW,), jnp.int32),
            pltpu.VMEM((3, CH, D), jnp.float32),
            pltpu.SemaphoreType.DMA((3,)),
        ],
        compiler_params=pltpu.CompilerParams(needs_layout_passes=False, use_tc_tiling_on_sc=False),
    )(tmap, table32)


def _tc1_body(tg_ref, agg_ref, out_ref):
    tu = tg_ref[...] + agg_ref[0] + agg_ref[1]
    out_ref[pl.ds(0, N_TISSUE), :] = tu
    out_ref[pl.ds(N_TISSUE, TAB - N_TISSUE), :] = jnp.zeros(
        (TAB - N_TISSUE, D), jnp.float32)


def _rep_body(tab_ref, out_ref):
    out_ref[...] = tab_ref[...]


def _tc2_body(cell_ref, tf_ref, wcq_ref, wtp_ref, b_ref, atd_ref, out_ref):
    cell = cell_ref[...]
    tf = tf_ref[...]
    cq = jnp.dot(cell.astype(jnp.bfloat16), wcq_ref[...],
                 preferred_element_type=jnp.float32)
    tk = jnp.dot(tf.astype(jnp.bfloat16), wtp_ref[...],
                 preferred_element_type=jnp.float32)
    h = jnp.tanh(cq + tk + b_ref[...])
    score = jnp.sum(h * atd_ref[...], axis=1, keepdims=True)
    wgt = jax.nn.sigmoid(score)
    out_ref[...] = cell + tf * wgt


def kernel(cell_graph, tissue_graph, cluster_assignments,
           W_cp, b_cp, W_tq, b_tq, a_bu,
           W_tp, b_tp, W_cq, b_cq, a_td):
    ca_flat = cluster_assignments.reshape(-1)
    ca_flat_pad = jnp.concatenate(
        [ca_flat, jnp.zeros((NPAD - N_CELL,), jnp.int32)])
    ca3d = ca_flat_pad.reshape(NW, NCH, CH)
    t_flat = jnp.repeat(jnp.arange(N_TISSUE, dtype=jnp.int32), C)
    t3d = jnp.concatenate(
        [t_flat, jnp.full((NPAD - N_CELL,), ZROW, jnp.int32)]).reshape(
            NW, NCH, CH)
    ca64f = jnp.pad(cluster_assignments, ((0, 0), (0, 64 - C)),
                    constant_values=-1).reshape(-1)
    neg1 = jnp.full((SPW,), -1, jnp.int32)
    ztab = jnp.zeros((TAB, D), jnp.float32)

    agg, tmap = _sc1(ca3d, t3d, cell_graph, ca64f, neg1, ztab)

    table = pl.pallas_call(
        _tc1_body,
        out_shape=jax.ShapeDtypeStruct((TAB, D), jnp.float32),
    )(tissue_graph, agg)

    table32 = pl.pallas_call(
        _rep_body,
        grid=(NW,),
        in_specs=[pl.BlockSpec((TAB, D), lambda i: (0, 0))],
        out_specs=pl.BlockSpec((TAB, D), lambda i: (i, 0)),
        out_shape=jax.ShapeDtypeStruct((NW * TAB, D), jnp.float32),
    )(table)

    tfeat = _sc2(tmap, table32)

    R = 1000
    bsum = (b_cq + b_tp).reshape(1, D)
    cell_updated = pl.pallas_call(
        _tc2_body,
        grid=(N_CELL // R,),
        in_specs=[
            pl.BlockSpec((R, D), lambda i: (i, 0)),
            pl.BlockSpec((R, D), lambda i: (i, 0)),
            pl.BlockSpec((D, D), lambda i: (0, 0)),
            pl.BlockSpec((D, D), lambda i: (0, 0)),
            pl.BlockSpec((1, D), lambda i: (0, 0)),
            pl.BlockSpec((1, D), lambda i: (0, 0)),
        ],
        out_specs=pl.BlockSpec((R, D), lambda i: (i, 0)),
        out_shape=jax.ShapeDtypeStruct((N_CELL, D), jnp.float32),
    )(cell_graph, tfeat, W_cq.T.astype(jnp.bfloat16),
      W_tp.T.astype(jnp.bfloat16), bsum, a_td)

    tissue_updated = table[:N_TISSUE]
    return (cell_updated, tissue_updated)


# speedup vs baseline: 1.5051x; 1.0676x over previous
import jax
import jax.numpy as jnp
from jax import lax
from jax.experimental import pallas as pl
from jax.experimental.pallas import tpu as pltpu
from jax.experimental.pallas import tpu_sc as plsc

N_CELL = 100000
N_TISSUE = 2000
C = 50
D = 128

NW = 32
SPW = 3136
NPAD = NW * SPW
CH = 112
NCH = SPW // CH
TAB = 2056
ZROW = 2048
TCHUNK = 200


def _sc1_body(ca3d, t3d, cell_hbm, ca64f, neg1, ztab,
              agg_hbm, tmap_hbm,
              idx_a, tix_a, rows_a, flat_v, buf_v, shared, gsem):
    c = lax.axis_index("c")
    s = lax.axis_index("s")
    w = c * 16 + s

    soff = pl.multiple_of(s * 128, 8)
    pltpu.sync_copy(ztab.at[pl.ds(soff, 128)], shared.at[pl.ds(soff, 128)])

    @pl.when(s == 15)
    def _():
        pltpu.sync_copy(ztab.at[pl.ds(2048, 8)], shared.at[pl.ds(2048, 8)])

    plsc.subcore_barrier()

    pltpu.sync_copy(ca3d.at[w], idx_a)
    pltpu.sync_copy(t3d.at[w], tix_a)

    cp0 = pltpu.async_copy(cell_hbm.at[idx_a.at[0]], rows_a.at[0], gsem.at[0])
    cp1 = pltpu.async_copy(cell_hbm.at[idx_a.at[1]], rows_a.at[1], gsem.at[1])

    def chunk_a(k, _):
        sl = lax.rem(k, 3)

        @pl.when(k + 2 < NCH)
        def _():
            nsl = lax.rem(k + 2, 3)
            pltpu.async_copy(cell_hbm.at[idx_a.at[k + 2]], rows_a.at[nsl],
                             gsem.at[nsl])

        pltpu.make_async_copy(cell_hbm.at[idx_a.at[k]], rows_a.at[sl],
                              gsem.at[sl]).wait()
        pltpu.sync_copy(rows_a.at[sl], shared.at[tix_a.at[k]], add=True)
        return 0

    lax.fori_loop(0, NCH, chunk_a, 0)
    del cp0, cp1
    plsc.subcore_barrier()

    @pl.when(s < 15)
    def _():
        pltpu.sync_copy(shared.at[pl.ds(soff, 128)],
                        agg_hbm.at[c, pl.ds(soff, 128)])

    @pl.when(s == 15)
    def _():
        pltpu.sync_copy(shared.at[pl.ds(1920, 80)],
                        agg_hbm.at[c, pl.ds(1920, 80)])

    pltpu.sync_copy(neg1, buf_v)
    cbase = w * SPW

    lanes = lax.iota(jnp.int32, 16)

    def tblock(rc, tv0):
        roff = pl.multiple_of(rc * TCHUNK * 64, 16)
        pltpu.sync_copy(ca64f.at[pl.ds(roff, TCHUNK * 64)], flat_v)

        def trow(j, carry):
            tv, ivs = carry
            vs = [plsc.load_gather(flat_v, [ivs[k]]) for k in range(4)]
            ds_ = [v - cbase for v in vs]
            ms = [(d >= 0) & (d < SPW) for d in ds_]
            for k in range(4):
                plsc.store_scatter(buf_v, [ds_[k]], tv, mask=ms[k])
            return tv + 1, tuple(iv + 64 for iv in ivs)

        ivs0 = tuple(lanes + k * 16 for k in range(4))
        tv1, _ = lax.fori_loop(0, TCHUNK, trow, (tv0, ivs0))
        return tv1

    tv0 = jnp.zeros((16,), jnp.int32)
    lax.fori_loop(0, N_TISSUE // TCHUNK, tblock, tv0)
    pltpu.sync_copy(buf_v, tmap_hbm.at[pl.ds(pl.multiple_of(w * SPW, 16), SPW)])


def _sc1(ca3d, t3d, cell_graph, ca64f, neg1, ztab):
    mesh = plsc.VectorSubcoreMesh(core_axis_name="c", subcore_axis_name="s")
    return pl.kernel(
        _sc1_body,
        out_type=[
            jax.ShapeDtypeStruct((2, N_TISSUE, D), jnp.float32),
            jax.ShapeDtypeStruct((NPAD,), jnp.int32),
        ],
        mesh=mesh,
        scratch_types=[
            pltpu.VMEM((NCH, CH), jnp.int32),
            pltpu.VMEM((NCH, CH), jnp.int32),
            pltpu.VMEM((3, CH, D), jnp.float32),
            pltpu.VMEM((TCHUNK * 64,), jnp.int32),
            pltpu.VMEM((SPW,), jnp.int32),
            pltpu.VMEM_SHARED((TAB, D), jnp.float32),
            pltpu.SemaphoreType.DMA((3,)),
        ],
        compiler_params=pltpu.CompilerParams(needs_layout_passes=False, use_tc_tiling_on_sc=False),
    )(ca3d, t3d, cell_graph, ca64f, neg1, ztab)


def _sc2_body(tmap_hbm, table_hbm, tfeat_hbm, idx_v, rows_b, sems):
    c = lax.axis_index("c")
    s = lax.axis_index("s")
    w = c * 16 + s

    pltpu.sync_copy(tmap_hbm.at[pl.ds(pl.multiple_of(w * SPW, 16), SPW)], idx_v)

    tb = w * TAB

    def clampb(h, iv):
        v = plsc.load_gather(idx_v, [iv])
        plsc.store_scatter(idx_v, [iv], jnp.where(v < 0, ZROW, v) + tb)
        return iv + 16

    lax.fori_loop(0, SPW // 16, clampb, lax.iota(jnp.int32, 16))

    cp0 = pltpu.async_copy(table_hbm.at[idx_v.at[pl.ds(0, CH)]], rows_b.at[0],
                           sems.at[0])
    cp1 = pltpu.async_copy(table_hbm.at[idx_v.at[pl.ds(CH, CH)]], rows_b.at[1],
                           sems.at[1])
    cp2 = pltpu.async_copy(table_hbm.at[idx_v.at[pl.ds(2 * CH, CH)]],
                           rows_b.at[2], sems.at[2])

    def chunk(k, _):
        sl = lax.rem(k, 4)

        @pl.when(k + 3 < NCH)
        def _():
            nsl = lax.rem(k + 3, 4)
            io = pl.multiple_of((k + 3) * CH, 16)
            pltpu.async_copy(table_hbm.at[idx_v.at[pl.ds(io, CH)]],
                             rows_b.at[nsl], sems.at[nsl])

        io2 = pl.multiple_of(k * CH, 16)
        pltpu.make_async_copy(table_hbm.at[idx_v.at[pl.ds(io2, CH)]],
                              rows_b.at[sl], sems.at[sl]).wait()
        base = pl.multiple_of(w * SPW + k * CH, 16)
        pltpu.sync_copy(rows_b.at[sl], tfeat_hbm.at[pl.ds(base, CH)])
        return 0

    lax.fori_loop(0, NCH, chunk, 0)
    del cp0, cp1, cp2


def _sc2(tmap, table32):
    mesh = plsc.VectorSubcoreMesh(core_axis_name="c", subcore_axis_name="s")
    return pl.kernel(
        _sc2_body,
        out_type=jax.ShapeDtypeStruct((NPAD, D), jnp.float32),
        mesh=mesh,
        scratch_types=[
            pltpu.VMEM((SPW,), jnp.int32),
            pltpu.VMEM((4, CH, D), jnp.float32),
            pltpu.SemaphoreType.DMA((4,)),
        ],
        compiler_params=pltpu.CompilerParams(needs_layout_passes=False, use_tc_tiling_on_sc=False),
    )(tmap, table32)


def _tc1_body(tg_ref, agg_ref, out_ref):
    tu = tg_ref[...] + agg_ref[0] + agg_ref[1]
    out_ref[pl.ds(0, N_TISSUE), :] = tu
    out_ref[pl.ds(N_TISSUE, TAB - N_TISSUE), :] = jnp.zeros(
        (TAB - N_TISSUE, D), jnp.float32)


def _rep_body(tab_ref, out_ref):
    out_ref[...] = tab_ref[...]


def _tc2_body(cell_ref, tf_ref, wcq_ref, wtp_ref, b_ref, atd_ref, out_ref):
    cell = cell_ref[...]
    tf = tf_ref[...]
    cq = jnp.dot(cell.astype(jnp.bfloat16), wcq_ref[...],
                 preferred_element_type=jnp.float32)
    tk = jnp.dot(tf.astype(jnp.bfloat16), wtp_ref[...],
                 preferred_element_type=jnp.float32)
    h = jnp.tanh(cq + tk + b_ref[...])
    score = jnp.sum(h * atd_ref[...], axis=1, keepdims=True)
    wgt = jax.nn.sigmoid(score)
    out_ref[...] = cell + tf * wgt


def kernel(cell_graph, tissue_graph, cluster_assignments,
           W_cp, b_cp, W_tq, b_tq, a_bu,
           W_tp, b_tp, W_cq, b_cq, a_td):
    ca_flat = cluster_assignments.reshape(-1)
    ca_flat_pad = jnp.concatenate(
        [ca_flat, jnp.zeros((NPAD - N_CELL,), jnp.int32)])
    ca3d = ca_flat_pad.reshape(NW, NCH, CH)
    t_flat = jnp.repeat(jnp.arange(N_TISSUE, dtype=jnp.int32), C)
    t3d = jnp.concatenate(
        [t_flat, jnp.full((NPAD - N_CELL,), ZROW, jnp.int32)]).reshape(
            NW, NCH, CH)
    ca64f = jnp.pad(cluster_assignments, ((0, 0), (0, 64 - C)),
                    constant_values=-1).reshape(-1)
    neg1 = jnp.full((SPW,), -1, jnp.int32)
    ztab = jnp.zeros((TAB, D), jnp.float32)

    agg, tmap = _sc1(ca3d, t3d, cell_graph, ca64f, neg1, ztab)

    table = pl.pallas_call(
        _tc1_body,
        out_shape=jax.ShapeDtypeStruct((TAB, D), jnp.float32),
    )(tissue_graph, agg)

    table32 = pl.pallas_call(
        _rep_body,
        grid=(NW,),
        in_specs=[pl.BlockSpec((TAB, D), lambda i: (0, 0))],
        out_specs=pl.BlockSpec((TAB, D), lambda i: (i, 0)),
        out_shape=jax.ShapeDtypeStruct((NW * TAB, D), jnp.float32),
    )(table)

    tfeat = _sc2(tmap, table32)

    R = 2000
    bsum = (b_cq + b_tp).reshape(1, D)
    cell_updated = pl.pallas_call(
        _tc2_body,
        grid=(N_CELL // R,),
        in_specs=[
            pl.BlockSpec((R, D), lambda i: (i, 0)),
            pl.BlockSpec((R, D), lambda i: (i, 0)),
            pl.BlockSpec((D, D), lambda i: (0, 0)),
            pl.BlockSpec((D, D), lambda i: (0, 0)),
            pl.BlockSpec((1, D), lambda i: (0, 0)),
            pl.BlockSpec((1, D), lambda i: (0, 0)),
        ],
        out_specs=pl.BlockSpec((R, D), lambda i: (i, 0)),
        out_shape=jax.ShapeDtypeStruct((N_CELL, D), jnp.float32),
    )(cell_graph, tfeat, W_cq.T.astype(jnp.bfloat16),
      W_tp.T.astype(jnp.bfloat16), bsum, a_td)

    tissue_updated = table[:N_TISSUE]
    return (cell_updated, tissue_updated)
